# baseline (speedup 1.0000x reference)
import functools

import jax
import jax.numpy as jnp
from jax import lax
from jax.experimental import pallas as pl
from jax.experimental.pallas import tpu as pltpu
from jax.experimental.pallas import tpu_sc as plsc

N = 10000
NP = 10240
D = 128
E = 320000
NTILES = 16
ET = E // NTILES
K = 80
NB = ET // K
KD = 2000
RT = NP // NTILES
RB = K


def _rsqrt16(d):
    d = jnp.maximum(d, jnp.float32(1e-12))
    half = d * jnp.float32(0.5)
    i = plsc.bitcast(d, jnp.int32)
    y = plsc.bitcast(jnp.int32(0x5F3759DF) - (i >> 1), jnp.float32)
    for _ in range(3):
        y = y * (jnp.float32(1.5) - half * y * y)
    return y


def _sc_body(x_ref, xb_ref, src_ref, dst_ref, ew_ref, agg_ref,
             sidx0, sidx1, didx0, didx1, ewb0, ewb1, dsc0, dsc1, normb,
             rows0, rows1, rbf0, rbf1, disl, dbuf, deg_s, agg_s,
             sem_i0, sem_i1, sem_g0, sem_g1, sem_s0, sem_s1):
    cid = lax.axis_index("c")
    sid = lax.axis_index("s")
    rbase = pl.multiple_of(sid * RT, 16)
    ebase = cid * E + sid * ET

    sidx = (sidx0, sidx1)
    didx = (didx0, didx1)
    ewb = (ewb0, ewb1)
    dsc = (dsc0, dsc1)
    rows = (rows0, rows1)
    rbf = (rbf0, rbf1)
    sem_i = (sem_i0, sem_i1)
    sem_g = (sem_g0, sem_g1)
    sem_s = (sem_s0, sem_s1)

    ones = jnp.ones((16,), jnp.float32)
    for i in range(RT // 16):
        dbuf[pl.ds(i * 16, 16)] = ones
    pltpu.sync_copy(dbuf, deg_s.at[pl.ds(rbase, RT)])
    plsc.subcore_barrier()

    def dld_start(t, s):
        base = pl.multiple_of(ebase + t * K, 16)
        pltpu.async_copy(dst_ref.at[pl.ds(base, K)], didx[s], sem_i[s])
        pltpu.async_copy(ew_ref.at[pl.ds(base, K)], ewb[s], sem_i[s])

    def dld_wait(s):
        for _ in range(2):
            pltpu.make_async_copy(
                src_ref.at[pl.ds(0, K)], didx[s], sem_i[s]).wait()

    def deg_block(t, s):
        has_next2 = not (isinstance(t, int) and t >= NB - 2)
        dld_wait(s)
        pltpu.sync_copy(ewb[s], deg_s.at[didx[s]], add=True)
        if has_next2:
            dld_start(t + 2, s)

    dld_start(0, 0)
    dld_start(1, 1)
    deg_block(0, 0)
    deg_block(1, 1)

    def deg_pair(p, _):
        t0 = 2 * p + 2
        deg_block(t0, 0)
        deg_block(t0 + 1, 1)
        return 0

    lax.fori_loop(0, (NB - 4) // 2, deg_pair, 0)
    deg_block(NB - 2, 0)
    deg_block(NB - 1, 1)
    plsc.subcore_barrier()

    pltpu.sync_copy(deg_s.at[pl.ds(rbase, RT)], dbuf)
    for i in range(RT // 16):
        dbuf[pl.ds(i * 16, 16)] = _rsqrt16(dbuf[pl.ds(i * 16, 16)])
    pltpu.sync_copy(dbuf, deg_s.at[pl.ds(rbase, RT)])
    plsc.subcore_barrier()
    pltpu.sync_copy(deg_s, disl)

    def x_start(rb, s):
        r0 = pl.multiple_of(sid * RT + rb * RB, 16)
        pltpu.async_copy(x_ref.at[pl.ds(r0, RB)], rows[s], sem_g[s])

    def x_wait(s):
        pltpu.make_async_copy(
            x_ref.at[pl.ds(0, RB)], rows[s], sem_g[s]).wait()

    x_start(0, 0)
    for rb in range(RT // RB):
        s = rb % 2
        x_wait(s)
        if rb + 1 < RT // RB:
            x_start(rb + 1, 1 - s)
        r0 = pl.multiple_of(sid * RT + rb * RB, 16)
        rows_s = rows[s]

        def init_grp(g, _, r0=r0, rows_s=rows_s):
            dv = disl[pl.ds(r0 + g * 16, 16)]
            dv = dv * dv
            for i in range(16):
                e = g * 16 + i
                s2 = dv[i]
                for j in range(D // 16):
                    rows_s[e, pl.ds(j * 16, 16)] = (
                        rows_s[e, pl.ds(j * 16, 16)] * s2)
            return 0

        lax.fori_loop(0, RB // 16, init_grp, 0)
        pltpu.sync_copy(rows_s, agg_s.at[pl.ds(r0, RB)])
    plsc.subcore_barrier()

    def idx_start(b, s):
        base = pl.multiple_of(ebase + b * K, 16)
        pltpu.async_copy(src_ref.at[pl.ds(base, K)], sidx[s], sem_i[s])
        pltpu.async_copy(dst_ref.at[pl.ds(base, K)], didx[s], sem_i[s])
        pltpu.async_copy(ew_ref.at[pl.ds(base, K)], ewb[s], sem_i[s])

    def idx_wait(s):
        for _ in range(3):
            pltpu.make_async_copy(
                src_ref.at[pl.ds(0, K)], sidx[s], sem_i[s]).wait()

    def gather_wait(s):
        pltpu.make_async_copy(xb_ref.at[sidx[s]], rbf[s], sem_g[s]).wait()

    def scatter_wait(s):
        pltpu.make_async_copy(rows[s], agg_s.at[dsc[s]], sem_s[s]).wait()

    def emit_block(b, s):
        o = 1 - s
        is_first = isinstance(b, int) and b == 0
        has_next = not (isinstance(b, int) and b >= NB - 1)
        has_next2 = not (isinstance(b, int) and b >= NB - 2)
        if not is_first:
            scatter_wait(o)
        if has_next:
            idx_wait(o)
            pltpu.async_copy(xb_ref.at[sidx[o]], rbf[o], sem_g[o])
        gather_wait(s)

        def norm_grp(g, _):
            sv = sidx[s][pl.ds(g * 16, 16)]
            dv = didx[s][pl.ds(g * 16, 16)]
            wv = ewb[s][pl.ds(g * 16, 16)]
            ns = plsc.load_gather(disl, [sv])
            nd = plsc.load_gather(disl, [dv])
            normb[pl.ds(g * 16, 16)] = ns * wv * nd
            dsc[s][pl.ds(g * 16, 16)] = dv
            return 0

        lax.fori_loop(0, K // 16, norm_grp, 0)

        def scale_grp(g, _):
            nv = normb[pl.ds(g * 16, 16)]
            for i in range(16):
                e = g * 16 + i
                sc = nv[i]
                for c in range(D // 32):
                    ch = plsc.bitcast(
                        rbf[s][e, pl.ds(c * 16, 16)], jnp.bfloat16)
                    lo, hi = plsc.unpack(
                        ch, format=plsc.PackFormat.INTERLEAVED)
                    rows[s][e, pl.ds(c * 32, 16)] = lo * sc
                    rows[s][e, pl.ds(c * 32 + 16, 16)] = hi * sc
            return 0

        lax.fori_loop(0, K // 16, scale_grp, 0)
        pltpu.async_copy(rows[s], agg_s.at[dsc[s]], sem_s[s], add=True)
        if has_next2:
            idx_start(b + 2, s)

    steady_start = 1 if (NB - 3) % 2 == 0 else 2
    idx_start(0, 0)
    idx_start(1, 1)
    idx_wait(0)
    pltpu.async_copy(xb_ref.at[sidx[0]], rbf[0], sem_g[0])
    for b in range(steady_start):
        emit_block(b, b % 2)

    def pair_body(p, _):
        b0 = 2 * p + steady_start
        emit_block(b0, steady_start % 2)
        emit_block(b0 + 1, 1 - steady_start % 2)
        return 0

    lax.fori_loop(0, (NB - 2 - steady_start) // 2, pair_body, 0)
    for b in (NB - 2, NB - 1):
        emit_block(b, b % 2)
    scatter_wait((NB - 1) % 2)
    plsc.subcore_barrier()

    pltpu.sync_copy(agg_s.at[pl.ds(rbase, RT)],
                    agg_ref.at[cid, pl.ds(rbase, RT)])


def _bf16_perm():
    import numpy as np
    perm = np.empty((D,), dtype=np.int32)
    for c in range(D // 32):
        for i in range(16):
            perm[32 * c + 2 * i] = 32 * c + i
            perm[32 * c + 2 * i + 1] = 32 * c + 16 + i
    return perm


_XB_PERM = _bf16_perm()


@jax.jit
def _sc_aggregate(x_pad, xb, src2, dst2, ew2):
    mesh = plsc.VectorSubcoreMesh(core_axis_name="c", subcore_axis_name="s")
    f = pl.kernel(
        _sc_body,
        out_type=jax.ShapeDtypeStruct((2, NP, D), jnp.float32),
        mesh=mesh,
        compiler_params=pltpu.CompilerParams(
            needs_layout_passes=False, use_tc_tiling_on_sc=False),
        scratch_types=[
            pltpu.VMEM((K,), jnp.int32),
            pltpu.VMEM((K,), jnp.int32),
            pltpu.VMEM((K,), jnp.int32),
            pltpu.VMEM((K,), jnp.int32),
            pltpu.VMEM((K,), jnp.float32),
            pltpu.VMEM((K,), jnp.float32),
            pltpu.VMEM((K,), jnp.int32),
            pltpu.VMEM((K,), jnp.int32),
            pltpu.VMEM((K,), jnp.float32),
            pltpu.VMEM((K, D), jnp.float32),
            pltpu.VMEM((K, D), jnp.float32),
            pltpu.VMEM((K, D // 2), jnp.int32),
            pltpu.VMEM((K, D // 2), jnp.int32),
            pltpu.VMEM((NP,), jnp.float32),
            pltpu.VMEM((RT,), jnp.float32),
            pltpu.VMEM_SHARED((NP,), jnp.float32),
            pltpu.VMEM_SHARED((NP, D), jnp.float32),
            pltpu.SemaphoreType.DMA,
            pltpu.SemaphoreType.DMA,
            pltpu.SemaphoreType.DMA,
            pltpu.SemaphoreType.DMA,
            pltpu.SemaphoreType.DMA,
            pltpu.SemaphoreType.DMA,
        ],
    )
    return f(x_pad, xb, src2, dst2, ew2)


def _tc_body(a_ref, wa_ref, wb_ref, ba_ref, bb_ref, o_ref):
    h = jnp.dot(a_ref[0], wa_ref[...], precision=lax.Precision.HIGHEST,
                preferred_element_type=jnp.float32)
    h += jnp.dot(a_ref[1], wb_ref[...], precision=lax.Precision.HIGHEST,
                 preferred_element_type=jnp.float32)
    o_ref[...] = h + ba_ref[...] + bb_ref[...]


@jax.jit
def _tc_matmul(agg, wa_t, wb_t, b_a, b_b):
    BM = 1024
    grid = (NP // BM,)
    return pl.pallas_call(
        _tc_body,
        grid=grid,
        in_specs=[
            pl.BlockSpec((2, BM, D), lambda i: (0, i, 0)),
            pl.BlockSpec((D, D), lambda i: (0, 0)),
            pl.BlockSpec((D, D), lambda i: (0, 0)),
            pl.BlockSpec((1, D), lambda i: (0, 0)),
            pl.BlockSpec((1, D), lambda i: (0, 0)),
        ],
        out_specs=pl.BlockSpec((BM, D), lambda i: (i, 0)),
        out_shape=jax.ShapeDtypeStruct((NP, D), jnp.float32),
    )(agg, wa_t, wb_t, b_a, b_b)


def kernel(x, edge_index_a, edge_weight_a, edge_index_b, edge_weight_b,
           W_a, b_a, W_b, b_b):
    x_pad = jnp.pad(x, ((0, NP - N), (0, 0)))
    xb = jax.lax.bitcast_convert_type(
        x_pad[:, _XB_PERM].astype(jnp.bfloat16).reshape(NP, D // 2, 2),
        jnp.int32)
    src2 = jnp.concatenate([edge_index_a[0], edge_index_b[0]])
    dst2 = jnp.concatenate([edge_index_a[1], edge_index_b[1]])
    ew2 = jnp.concatenate([edge_weight_a, edge_weight_b])
    agg = _sc_aggregate(x_pad, xb, src2, dst2, ew2)
    out = _tc_matmul(agg, W_a.T, W_b.T,
                     b_a.reshape(1, D), b_b.reshape(1, D))
    return out[:N]


# speedup vs baseline: 1.8618x; 1.8618x over previous
import functools

import jax
import jax.numpy as jnp
from jax import lax
from jax.experimental import pallas as pl
from jax.experimental.pallas import tpu as pltpu
from jax.experimental.pallas import tpu_sc as plsc

N = 10000
NP = 10240
D = 128
E = 320000
NTILES = 16
ET = E // NTILES
K = 80
NB = ET // K
KD = 2000
RT = NP // NTILES
RB = K


def _rsqrt16(d):
    d = jnp.maximum(d, jnp.float32(1e-12))
    half = d * jnp.float32(0.5)
    i = plsc.bitcast(d, jnp.int32)
    y = plsc.bitcast(jnp.int32(0x5F3759DF) - (i >> 1), jnp.float32)
    for _ in range(3):
        y = y * (jnp.float32(1.5) - half * y * y)
    return y


def _sc_body(x_ref, src_ref, dst_ref, ew_ref, agg_ref,
             sidx0, sidx1, didx0, didx1, ewb0, ewb1, dsc0, dsc1, normb,
             rows0, rows1, disl, dbuf, deg_s, agg_s,
             sem_i0, sem_i1, sem_g0, sem_g1, sem_s0, sem_s1):
    cid = lax.axis_index("c")
    sid = lax.axis_index("s")
    rbase = pl.multiple_of(sid * RT, 16)
    ebase = cid * E + sid * ET

    sidx = (sidx0, sidx1)
    didx = (didx0, didx1)
    ewb = (ewb0, ewb1)
    dsc = (dsc0, dsc1)
    rows = (rows0, rows1)
    sem_i = (sem_i0, sem_i1)
    sem_g = (sem_g0, sem_g1)
    sem_s = (sem_s0, sem_s1)

    ones = jnp.ones((16,), jnp.float32)
    for i in range(RT // 16):
        dbuf[pl.ds(i * 16, 16)] = ones
    pltpu.sync_copy(dbuf, deg_s.at[pl.ds(rbase, RT)])
    plsc.subcore_barrier()

    def dld_start(t, s):
        base = pl.multiple_of(ebase + t * K, 16)
        pltpu.async_copy(dst_ref.at[pl.ds(base, K)], didx[s], sem_i[s])
        pltpu.async_copy(ew_ref.at[pl.ds(base, K)], ewb[s], sem_i[s])

    def dld_wait(s):
        for _ in range(2):
            pltpu.make_async_copy(
                src_ref.at[pl.ds(0, K)], didx[s], sem_i[s]).wait()

    def deg_block(t, s):
        has_next2 = not (isinstance(t, int) and t >= NB - 2)
        dld_wait(s)
        pltpu.sync_copy(ewb[s], deg_s.at[didx[s]], add=True)
        if has_next2:
            dld_start(t + 2, s)

    dld_start(0, 0)
    dld_start(1, 1)
    deg_block(0, 0)
    deg_block(1, 1)

    def deg_pair(p, _):
        t0 = 2 * p + 2
        deg_block(t0, 0)
        deg_block(t0 + 1, 1)
        return 0

    lax.fori_loop(0, (NB - 4) // 2, deg_pair, 0)
    deg_block(NB - 2, 0)
    deg_block(NB - 1, 1)
    plsc.subcore_barrier()

    pltpu.sync_copy(deg_s.at[pl.ds(rbase, RT)], dbuf)
    for i in range(RT // 16):
        dbuf[pl.ds(i * 16, 16)] = _rsqrt16(dbuf[pl.ds(i * 16, 16)])
    pltpu.sync_copy(dbuf, deg_s.at[pl.ds(rbase, RT)])
    plsc.subcore_barrier()
    pltpu.sync_copy(deg_s, disl)

    def x_start(rb, s):
        r0 = pl.multiple_of(sid * RT + rb * RB, 16)
        pltpu.async_copy(x_ref.at[pl.ds(r0, RB)], rows[s], sem_g[s])

    def x_wait(s):
        pltpu.make_async_copy(
            x_ref.at[pl.ds(0, RB)], rows[s], sem_g[s]).wait()

    x_start(0, 0)
    for rb in range(RT // RB):
        s = rb % 2
        x_wait(s)
        if rb + 1 < RT // RB:
            x_start(rb + 1, 1 - s)
        r0 = pl.multiple_of(sid * RT + rb * RB, 16)
        rows_s = rows[s]

        def init_grp(g, _, r0=r0, rows_s=rows_s):
            dv = disl[pl.ds(r0 + g * 16, 16)]
            dv = dv * dv
            for i in range(16):
                e = g * 16 + i
                s2 = dv[i]
                for j in range(D // 16):
                    rows_s[e, pl.ds(j * 16, 16)] = (
                        rows_s[e, pl.ds(j * 16, 16)] * s2)
            return 0

        lax.fori_loop(0, RB // 16, init_grp, 0)
        pltpu.sync_copy(rows_s, agg_s.at[pl.ds(r0, RB)])
    plsc.subcore_barrier()

    def idx_start(b, s):
        base = pl.multiple_of(ebase + b * K, 16)
        pltpu.async_copy(src_ref.at[pl.ds(base, K)], sidx[s], sem_i[s])
        pltpu.async_copy(dst_ref.at[pl.ds(base, K)], didx[s], sem_i[s])
        pltpu.async_copy(ew_ref.at[pl.ds(base, K)], ewb[s], sem_i[s])

    def idx_wait(s):
        for _ in range(3):
            pltpu.make_async_copy(
                src_ref.at[pl.ds(0, K)], sidx[s], sem_i[s]).wait()

    def gather_wait(s):
        pltpu.make_async_copy(x_ref.at[sidx[s]], rows[s], sem_g[s]).wait()

    def scatter_wait(s):
        pltpu.make_async_copy(rows[s], agg_s.at[dsc[s]], sem_s[s]).wait()

    def emit_block(b, s):
        o = 1 - s
        is_first = isinstance(b, int) and b == 0
        has_next = not (isinstance(b, int) and b >= NB - 1)
        has_next2 = not (isinstance(b, int) and b >= NB - 2)
        if not is_first:
            scatter_wait(o)
        if has_next:
            idx_wait(o)
            pltpu.async_copy(x_ref.at[sidx[o]], rows[o], sem_g[o])
        gather_wait(s)

        def norm_grp(g, _):
            sv = sidx[s][pl.ds(g * 16, 16)]
            dv = didx[s][pl.ds(g * 16, 16)]
            wv = ewb[s][pl.ds(g * 16, 16)]
            ns = plsc.load_gather(disl, [sv])
            nd = plsc.load_gather(disl, [dv])
            normb[pl.ds(g * 16, 16)] = ns * wv * nd
            dsc[s][pl.ds(g * 16, 16)] = dv
            return 0

        lax.fori_loop(0, K // 16, norm_grp, 0)

        def scale_grp(g, _):
            nv = normb[pl.ds(g * 16, 16)]
            for i in range(16):
                e = g * 16 + i
                sc = nv[i]
                for j in range(D // 16):
                    rows[s][e, pl.ds(j * 16, 16)] = (
                        rows[s][e, pl.ds(j * 16, 16)] * sc)
            return 0

        lax.fori_loop(0, K // 16, scale_grp, 0)
        pltpu.async_copy(rows[s], agg_s.at[dsc[s]], sem_s[s], add=True)
        if has_next2:
            idx_start(b + 2, s)

    steady_start = 1 if (NB - 3) % 2 == 0 else 2
    idx_start(0, 0)
    idx_start(1, 1)
    idx_wait(0)
    pltpu.async_copy(x_ref.at[sidx[0]], rows[0], sem_g[0])
    for b in range(steady_start):
        emit_block(b, b % 2)

    def pair_body(p, _):
        b0 = 2 * p + steady_start
        emit_block(b0, steady_start % 2)
        emit_block(b0 + 1, 1 - steady_start % 2)
        return 0

    lax.fori_loop(0, (NB - 2 - steady_start) // 2, pair_body, 0)
    for b in (NB - 2, NB - 1):
        emit_block(b, b % 2)
    scatter_wait((NB - 1) % 2)
    plsc.subcore_barrier()

    pltpu.sync_copy(agg_s.at[pl.ds(rbase, RT)],
                    agg_ref.at[cid, pl.ds(rbase, RT)])


@jax.jit
def _sc_aggregate(x_pad, src2, dst2, ew2):
    mesh = plsc.VectorSubcoreMesh(core_axis_name="c", subcore_axis_name="s")
    f = pl.kernel(
        _sc_body,
        out_type=jax.ShapeDtypeStruct((2, NP, D), jnp.float32),
        mesh=mesh,
        compiler_params=pltpu.CompilerParams(needs_layout_passes=False),
        scratch_types=[
            pltpu.VMEM((K,), jnp.int32),
            pltpu.VMEM((K,), jnp.int32),
            pltpu.VMEM((K,), jnp.int32),
            pltpu.VMEM((K,), jnp.int32),
            pltpu.VMEM((K,), jnp.float32),
            pltpu.VMEM((K,), jnp.float32),
            pltpu.VMEM((K,), jnp.int32),
            pltpu.VMEM((K,), jnp.int32),
            pltpu.VMEM((K,), jnp.float32),
            pltpu.VMEM((K, D), jnp.float32),
            pltpu.VMEM((K, D), jnp.float32),
            pltpu.VMEM((NP,), jnp.float32),
            pltpu.VMEM((RT,), jnp.float32),
            pltpu.VMEM_SHARED((NP,), jnp.float32),
            pltpu.VMEM_SHARED((NP, D), jnp.float32),
            pltpu.SemaphoreType.DMA,
            pltpu.SemaphoreType.DMA,
            pltpu.SemaphoreType.DMA,
            pltpu.SemaphoreType.DMA,
            pltpu.SemaphoreType.DMA,
            pltpu.SemaphoreType.DMA,
        ],
    )
    return f(x_pad, src2, dst2, ew2)


def _tc_body(a_ref, wa_ref, wb_ref, ba_ref, bb_ref, o_ref):
    h = jnp.dot(a_ref[0], wa_ref[...], precision=lax.Precision.HIGHEST,
                preferred_element_type=jnp.float32)
    h += jnp.dot(a_ref[1], wb_ref[...], precision=lax.Precision.HIGHEST,
                 preferred_element_type=jnp.float32)
    o_ref[...] = h + ba_ref[...] + bb_ref[...]


@jax.jit
def _tc_matmul(agg, wa_t, wb_t, b_a, b_b):
    BM = 1024
    grid = (NP // BM,)
    return pl.pallas_call(
        _tc_body,
        grid=grid,
        in_specs=[
            pl.BlockSpec((2, BM, D), lambda i: (0, i, 0)),
            pl.BlockSpec((D, D), lambda i: (0, 0)),
            pl.BlockSpec((D, D), lambda i: (0, 0)),
            pl.BlockSpec((1, D), lambda i: (0, 0)),
            pl.BlockSpec((1, D), lambda i: (0, 0)),
        ],
        out_specs=pl.BlockSpec((BM, D), lambda i: (i, 0)),
        out_shape=jax.ShapeDtypeStruct((NP, D), jnp.float32),
    )(agg, wa_t, wb_t, b_a, b_b)


def kernel(x, edge_index_a, edge_weight_a, edge_index_b, edge_weight_b,
           W_a, b_a, W_b, b_b):
    x_pad = jnp.pad(x, ((0, NP - N), (0, 0)))
    src2 = jnp.concatenate([edge_index_a[0], edge_index_b[0]])
    dst2 = jnp.concatenate([edge_index_a[1], edge_index_b[1]])
    ew2 = jnp.concatenate([edge_weight_a, edge_weight_b])
    agg = _sc_aggregate(x_pad, src2, dst2, ew2)
    out = _tc_matmul(agg, W_a.T, W_b.T,
                     b_a.reshape(1, D), b_b.reshape(1, D))
    return out[:N]
